# baseline (speedup 1.0000x reference)
import functools

import jax
import jax.numpy as jnp
from jax import lax
from jax.experimental import pallas as pl
from jax.experimental.pallas import tpu as pltpu
from jax.experimental.pallas import tpu_sc as plsc

N_NEURONS = 4096
N_LANES = 16

_f32 = jnp.float32
_i32 = jnp.int32


def _pick_chunk(epw: int) -> int:
    for c in range(4000, 15, -16):
        if epw % c == 0:
            return c
    raise ValueError(f"edges per worker {epw} not divisible by a usable chunk")


@functools.lru_cache(maxsize=None)
def _make_sc_edges(n_edges: int, batch: int, mode: int, input_size: int):
    info = plsc.get_sparse_core_info()
    nc, ns = info.num_cores, info.num_subcores
    nw = nc * ns
    assert n_edges % nw == 0, (n_edges, nw)
    epw = n_edges // nw
    chunk = _pick_chunk(epw)
    n_chunks = epw // chunk
    n_groups = chunk // N_LANES
    unroll = next(u for u in (25, 10, 5, 4, 2, 1) if n_groups % u == 0)
    assert n_chunks % 2 == 0, n_chunks
    flat = N_NEURONS * batch
    assert flat % (N_LANES * 8) == 0, flat

    mesh = plsc.VectorSubcoreMesh(core_axis_name="c", subcore_axis_name="s")

    @functools.partial(
        pl.kernel,
        out_type=jax.ShapeDtypeStruct((nw, flat), _f32),
        mesh=mesh,
        compiler_params=pltpu.CompilerParams(needs_layout_passes=False),
        scratch_types=[
            pltpu.VMEM((flat,), _f32),
            pltpu.VMEM((flat // 2,), _f32),
            pltpu.VMEM((flat // 2,), _f32),
            pltpu.VMEM((chunk,), _i32),
            pltpu.VMEM((chunk,), _i32),
            pltpu.VMEM((chunk,), _i32),
            pltpu.VMEM((chunk,), _i32),
            pltpu.VMEM((chunk,), _f32),
            pltpu.VMEM((chunk,), _f32),
            pltpu.SemaphoreType.DMA,
            pltpu.SemaphoreType.DMA,
            pltpu.VMEM((chunk + 2 * N_LANES,), _i32),
            pltpu.VMEM((chunk + 2 * N_LANES,), _i32),
            pltpu.VMEM((chunk + 2 * N_LANES,), _f32),
        ],
    )
    def sc_edges(a_hbm, f_hbm, t_hbm, w_hbm, o_hbm, a_v, o_va, o_vb,
                 f_v0, f_v1, t_v0, t_v1, w_v0, w_v1, sem0, sem1,
                 fc_v, tc_v, wc_v):
        cid = lax.axis_index("c")
        sid = lax.axis_index("s")
        wid = sid * nc + cid

        pltpu.sync_copy(a_hbm, a_v)

        zero16 = jnp.zeros((N_LANES,), _f32)

        def zero_body(i):
            base = i * (N_LANES * 8)
            for u in range(8):
                o_va[pl.ds(base + u * N_LANES, N_LANES)] = zero16
                o_vb[pl.ds(base + u * N_LANES, N_LANES)] = zero16

        plsc.parallel_loop(0, flat // (2 * N_LANES * 8))(zero_body)

        ebase = wid * epw
        bufs = ((f_v0, t_v0, w_v0, sem0), (f_v1, t_v1, w_v1, sem1))

        def issue(c, k):
            fk, tk, wk, sem = bufs[k]
            b0 = ebase + c * chunk
            pltpu.async_copy(f_hbm.at[pl.ds(b0, chunk)], fk, sem)
            pltpu.async_copy(t_hbm.at[pl.ds(b0, chunk)], tk, sem)
            pltpu.async_copy(w_hbm.at[pl.ds(b0, chunk)], wk, sem)

        def drain(k):
            fk, tk, wk, sem = bufs[k]
            pltpu.make_async_copy(f_hbm.at[pl.ds(0, chunk)], fk, sem).wait()
            pltpu.make_async_copy(t_hbm.at[pl.ds(0, chunk)], tk, sem).wait()
            pltpu.make_async_copy(w_hbm.at[pl.ds(0, chunk)], wk, sem).wait()

        half = batch // 2

        def group16(fref, tref, wref, off):
            f16 = fref[pl.ds(off, N_LANES)]
            t16 = tref[pl.ds(off, N_LANES)]
            w16 = wref[pl.ds(off, N_LANES)]
            for b in range(half):
                vals = plsc.load_gather(a_v, [f16 + (b * N_NEURONS)])
                plsc.addupdate_scatter(o_va, [t16 + (b * N_NEURONS)], w16 * vals)
                vals = plsc.load_gather(a_v, [f16 + ((b + half) * N_NEURONS)])
                plsc.addupdate_scatter(o_vb, [t16 + (b * N_NEURONS)], w16 * vals)

        def process_all(k):
            fk, tk, wk, _ = bufs[k]

            def group_body(g):
                base = g * (N_LANES * unroll)
                for u in range(unroll):
                    group16(fk, tk, wk, base + u * N_LANES)

            plsc.parallel_loop(0, n_groups // unroll)(group_body)

        lanes = lax.iota(_i32, N_LANES)
        zero16i = jnp.zeros((N_LANES,), _i32)

        def process_filtered(k):
            fk, tk, wk, _ = bufs[k]

            def comp_one(ncv, off):
                f16 = fk[pl.ds(off, N_LANES)]
                t16 = tk[pl.ds(off, N_LANES)]
                w16 = wk[pl.ds(off, N_LANES)]
                if mode == 1:
                    m = f16 < input_size
                else:
                    m = t16 >= (N_NEURONS - input_size)
                pos = ncv + plsc.cumsum(m.astype(_i32)) - 1
                plsc.store_scatter(fc_v, [pos], f16, mask=m)
                plsc.store_scatter(tc_v, [pos], t16, mask=m)
                plsc.store_scatter(wc_v, [pos], w16, mask=m)
                return ncv + plsc.all_reduce_population_count(m)

            cu = next(u for u in (5, 4, 2, 1) if n_groups % u == 0)

            def comp_body(g, ncv):
                base = g * (N_LANES * cu)
                for u in range(cu):
                    ncv = comp_one(ncv, base + u * N_LANES)
                return ncv

            ncv = plsc.parallel_loop(0, n_groups // cu, carry=zero16i)(comp_body)

            for p in range(2):
                pad_pos = ncv + lanes + (p * N_LANES)
                plsc.store_scatter(fc_v, [pad_pos], zero16i)
                plsc.store_scatter(tc_v, [pad_pos], zero16i)
                plsc.store_scatter(wc_v, [pad_pos], jnp.zeros((N_LANES,), _f32))

            nkept = jnp.max(ncv)
            n_kept_pairs = lax.shift_right_logical(nkept + (2 * N_LANES - 1), 5)

            def pbody(g):
                group16(fc_v, tc_v, wc_v, g * (2 * N_LANES))
                group16(fc_v, tc_v, wc_v, g * (2 * N_LANES) + N_LANES)

            plsc.parallel_loop(0, n_kept_pairs, unroll=2)(pbody)

        process = process_all if mode == 0 else process_filtered

        issue(0, 0)

        def pair_body(c2, _):
            c0 = 2 * c2
            issue(c0 + 1, 1)
            drain(0)
            process(0)

            @pl.when(c0 + 2 < n_chunks)
            def _():
                issue(c0 + 2, 0)

            drain(1)
            process(1)
            return 0

        lax.fori_loop(0, n_chunks // 2, pair_body, 0)

        pltpu.sync_copy(o_va, o_hbm.at[wid, pl.ds(0, flat // 2)])
        pltpu.sync_copy(o_vb, o_hbm.at[wid, pl.ds(flat // 2, flat // 2)])

    return sc_edges


@functools.lru_cache(maxsize=None)
def _make_tc_combine(nw: int, flat: int):
    def body(o_ref, a_ref):
        a_ref[...] = jnp.tanh(jnp.sum(o_ref[...], axis=0))

    return pl.pallas_call(
        body,
        out_shape=jax.ShapeDtypeStruct((flat,), _f32),
    )


def kernel(input_data, connection_weights, connection_indices, steps):
    batch, input_size = input_data.shape
    n_edges = connection_weights.shape[0]
    flat = N_NEURONS * batch

    sc_first = _make_sc_edges(n_edges, batch, 1, input_size)
    sc_mid = _make_sc_edges(n_edges, batch, 0, input_size)
    sc_last = _make_sc_edges(n_edges, batch, 2, input_size)
    info = plsc.get_sparse_core_info()
    nw = info.num_cores * info.num_subcores
    tc_combine = _make_tc_combine(nw, flat)

    a0 = jnp.zeros((batch, N_NEURONS), _f32)
    a0 = a0.at[:, :input_size].set(input_data)
    a0 = a0.reshape(flat)

    from_idx = connection_indices[0]
    to_idx = connection_indices[1]

    def step_body(k, a):
        sel = jnp.where(k == 0, 0, jnp.where(k == steps - 1, 2, 1))
        parts = lax.switch(
            sel,
            [sc_first, sc_mid, sc_last],
            a, from_idx, to_idx, connection_weights,
        )
        return tc_combine(parts)

    def run_generic(a):
        return lax.fori_loop(0, steps, step_body, a)

    def run_two(a):
        a1 = tc_combine(sc_first(a, from_idx, to_idx, connection_weights))
        return tc_combine(sc_last(a1, from_idx, to_idx, connection_weights))

    a_final = lax.cond(steps == 2, run_two, run_generic, a0)

    return a_final.reshape(batch, N_NEURONS)[:, -input_size:]


# speedup vs baseline: 1.0088x; 1.0088x over previous
import functools

import jax
import jax.numpy as jnp
from jax import lax
from jax.experimental import pallas as pl
from jax.experimental.pallas import tpu as pltpu
from jax.experimental.pallas import tpu_sc as plsc

N_NEURONS = 4096
N_LANES = 16

_f32 = jnp.float32
_i32 = jnp.int32


def _pick_chunk(epw: int) -> int:
    for c in range(4000, 15, -16):
        if epw % c == 0:
            return c
    raise ValueError(f"edges per worker {epw} not divisible by a usable chunk")


@functools.lru_cache(maxsize=None)
def _make_sc_edges(n_edges: int, batch: int, mode: int, input_size: int):
    info = plsc.get_sparse_core_info()
    nc, ns = info.num_cores, info.num_subcores
    nw = nc * ns
    assert n_edges % nw == 0, (n_edges, nw)
    epw = n_edges // nw
    chunk = _pick_chunk(epw)
    n_chunks = epw // chunk
    n_groups = chunk // N_LANES
    unroll = next(u for u in (25, 10, 5, 4, 2, 1) if n_groups % u == 0)
    assert n_chunks % 2 == 0, n_chunks
    flat = N_NEURONS * batch
    assert flat % (N_LANES * 8) == 0, flat

    mesh = plsc.VectorSubcoreMesh(core_axis_name="c", subcore_axis_name="s")

    @functools.partial(
        pl.kernel,
        out_type=jax.ShapeDtypeStruct((nw, flat), _f32),
        mesh=mesh,
        compiler_params=pltpu.CompilerParams(needs_layout_passes=False),
        scratch_types=[
            pltpu.VMEM((flat,), _f32),
            pltpu.VMEM((flat // 2,), _f32),
            pltpu.VMEM((flat // 2,), _f32),
            pltpu.VMEM((chunk,), _i32),
            pltpu.VMEM((chunk,), _i32),
            pltpu.VMEM((chunk,), _i32),
            pltpu.VMEM((chunk,), _i32),
            pltpu.VMEM((chunk,), _f32),
            pltpu.VMEM((chunk,), _f32),
            pltpu.SemaphoreType.DMA,
            pltpu.SemaphoreType.DMA,
            pltpu.VMEM((chunk + 2 * N_LANES,), _i32),
            pltpu.VMEM((chunk + 2 * N_LANES,), _i32),
            pltpu.VMEM((chunk + 2 * N_LANES,), _f32),
        ],
    )
    def sc_edges(a_hbm, f_hbm, t_hbm, w_hbm, o_hbm, a_v, o_va, o_vb,
                 f_v0, f_v1, t_v0, t_v1, w_v0, w_v1, sem0, sem1,
                 fc_v, tc_v, wc_v):
        cid = lax.axis_index("c")
        sid = lax.axis_index("s")
        wid = sid * nc + cid

        pltpu.sync_copy(a_hbm, a_v)

        zero16 = jnp.zeros((N_LANES,), _f32)

        def zero_body(i, _):
            base = i * (N_LANES * 8)
            for u in range(8):
                o_va[pl.ds(base + u * N_LANES, N_LANES)] = zero16
                o_vb[pl.ds(base + u * N_LANES, N_LANES)] = zero16
            return 0

        lax.fori_loop(0, flat // (2 * N_LANES * 8), zero_body, 0)

        ebase = wid * epw
        bufs = ((f_v0, t_v0, w_v0, sem0), (f_v1, t_v1, w_v1, sem1))

        def issue(c, k):
            fk, tk, wk, sem = bufs[k]
            b0 = ebase + c * chunk
            pltpu.async_copy(f_hbm.at[pl.ds(b0, chunk)], fk, sem)
            pltpu.async_copy(t_hbm.at[pl.ds(b0, chunk)], tk, sem)
            pltpu.async_copy(w_hbm.at[pl.ds(b0, chunk)], wk, sem)

        def drain(k):
            fk, tk, wk, sem = bufs[k]
            pltpu.make_async_copy(f_hbm.at[pl.ds(0, chunk)], fk, sem).wait()
            pltpu.make_async_copy(t_hbm.at[pl.ds(0, chunk)], tk, sem).wait()
            pltpu.make_async_copy(w_hbm.at[pl.ds(0, chunk)], wk, sem).wait()

        half = batch // 2

        def group16(fref, tref, wref, off):
            f16 = fref[pl.ds(off, N_LANES)]
            t16 = tref[pl.ds(off, N_LANES)]
            w16 = wref[pl.ds(off, N_LANES)]
            for b in range(half):
                vals = plsc.load_gather(a_v, [f16 + (b * N_NEURONS)])
                plsc.addupdate_scatter(o_va, [t16 + (b * N_NEURONS)], w16 * vals)
                vals = plsc.load_gather(a_v, [f16 + ((b + half) * N_NEURONS)])
                plsc.addupdate_scatter(o_vb, [t16 + (b * N_NEURONS)], w16 * vals)

        def process_all(k):
            fk, tk, wk, _ = bufs[k]

            def group_body(g):
                base = g * (N_LANES * unroll)
                for u in range(unroll):
                    group16(fk, tk, wk, base + u * N_LANES)

            plsc.parallel_loop(0, n_groups // unroll)(group_body)

        lanes = lax.iota(_i32, N_LANES)
        zero16i = jnp.zeros((N_LANES,), _i32)

        def process_filtered(k):
            fk, tk, wk, _ = bufs[k]

            def comp_one(ncv, off):
                f16 = fk[pl.ds(off, N_LANES)]
                t16 = tk[pl.ds(off, N_LANES)]
                w16 = wk[pl.ds(off, N_LANES)]
                if mode == 1:
                    m = f16 < input_size
                else:
                    m = t16 >= (N_NEURONS - input_size)
                pos = ncv + plsc.cumsum(m.astype(_i32)) - 1
                plsc.store_scatter(fc_v, [pos], f16, mask=m)
                plsc.store_scatter(tc_v, [pos], t16, mask=m)
                plsc.store_scatter(wc_v, [pos], w16, mask=m)
                return ncv + plsc.all_reduce_population_count(m)

            cu = next(u for u in (5, 4, 2, 1) if n_groups % u == 0)

            def comp_body(g, ncv):
                base = g * (N_LANES * cu)
                for u in range(cu):
                    ncv = comp_one(ncv, base + u * N_LANES)
                return ncv

            ncv = plsc.parallel_loop(0, n_groups // cu, carry=zero16i)(comp_body)

            for p in range(2):
                pad_pos = ncv + lanes + (p * N_LANES)
                plsc.store_scatter(fc_v, [pad_pos], zero16i)
                plsc.store_scatter(tc_v, [pad_pos], zero16i)
                plsc.store_scatter(wc_v, [pad_pos], jnp.zeros((N_LANES,), _f32))

            nkept = jnp.max(ncv)
            n_kept_pairs = lax.shift_right_logical(nkept + (2 * N_LANES - 1), 5)

            def pbody(g):
                group16(fc_v, tc_v, wc_v, g * (2 * N_LANES))
                group16(fc_v, tc_v, wc_v, g * (2 * N_LANES) + N_LANES)

            plsc.parallel_loop(0, n_kept_pairs)(pbody)

        process = process_all if mode == 0 else process_filtered

        issue(0, 0)

        def pair_body(c2, _):
            c0 = 2 * c2
            issue(c0 + 1, 1)
            drain(0)
            process(0)

            @pl.when(c0 + 2 < n_chunks)
            def _():
                issue(c0 + 2, 0)

            drain(1)
            process(1)
            return 0

        lax.fori_loop(0, n_chunks // 2, pair_body, 0)

        pltpu.sync_copy(o_va, o_hbm.at[wid, pl.ds(0, flat // 2)])
        pltpu.sync_copy(o_vb, o_hbm.at[wid, pl.ds(flat // 2, flat // 2)])

    return sc_edges


@functools.lru_cache(maxsize=None)
def _make_tc_combine(nw: int, flat: int):
    def body(o_ref, a_ref):
        a_ref[...] = jnp.tanh(jnp.sum(o_ref[...], axis=0))

    return pl.pallas_call(
        body,
        out_shape=jax.ShapeDtypeStruct((flat,), _f32),
    )


def kernel(input_data, connection_weights, connection_indices, steps):
    batch, input_size = input_data.shape
    n_edges = connection_weights.shape[0]
    flat = N_NEURONS * batch

    sc_first = _make_sc_edges(n_edges, batch, 1, input_size)
    sc_mid = _make_sc_edges(n_edges, batch, 0, input_size)
    sc_last = _make_sc_edges(n_edges, batch, 2, input_size)
    info = plsc.get_sparse_core_info()
    nw = info.num_cores * info.num_subcores
    tc_combine = _make_tc_combine(nw, flat)

    a0 = jnp.zeros((batch, N_NEURONS), _f32)
    a0 = a0.at[:, :input_size].set(input_data)
    a0 = a0.reshape(flat)

    from_idx = connection_indices[0]
    to_idx = connection_indices[1]

    def step_body(k, a):
        sel = jnp.where(k == 0, 0, jnp.where(k == steps - 1, 2, 1))
        parts = lax.switch(
            sel,
            [sc_first, sc_mid, sc_last],
            a, from_idx, to_idx, connection_weights,
        )
        return tc_combine(parts)

    def run_generic(a):
        return lax.fori_loop(0, steps, step_body, a)

    def run_two(a):
        a1 = tc_combine(sc_first(a, from_idx, to_idx, connection_weights))
        return tc_combine(sc_last(a1, from_idx, to_idx, connection_weights))

    a_final = lax.cond(steps == 2, run_two, run_generic, a0)

    return a_final.reshape(batch, N_NEURONS)[:, -input_size:]
